# baseline (device time: 48741 ns/iter reference)
import jax
import jax.numpy as jnp
from jax import lax
from jax.experimental import pallas as pl
from jax.experimental.pallas import tpu as pltpu

N_DEV = 4
B, Sq, Skv, Dh = 2, 256, 256, 64
H_LOC = 4
D_MODEL = 512
BLK = 64


def kernel(x, Wq, K_ext, V_ext, Wo):
    my = lax.axis_index("i")
    Kh = lax.dynamic_slice_in_dim(K_ext, my * H_LOC, H_LOC, axis=2)
    Vh = lax.dynamic_slice_in_dim(V_ext, my * H_LOC, H_LOC, axis=2)
    Kh = jnp.transpose(Kh, (0, 2, 1, 3))
    Vh = jnp.transpose(Vh, (0, 2, 1, 3))
    x2d = x.reshape(B * Sq, D_MODEL)

    def body(x_ref, wq_ref, k_ref, v_ref, wo_ref, out_ref,
             ctx_ref, comm_ref, send_sems, recv_sems):
        my_pos = lax.axis_index("i")
        left = lax.rem(my_pos + N_DEV - 1, N_DEV)
        right = lax.rem(my_pos + 1, N_DEV)

        barrier_sem = pltpu.get_barrier_semaphore()
        for nbr in (left, right):
            pl.semaphore_signal(
                barrier_sem, inc=1,
                device_id=(nbr,), device_id_type=pl.DeviceIdType.MESH,
            )
        pl.semaphore_wait(barrier_sem, 2)

        q2d = jnp.dot(x_ref[:, :], wq_ref[:, :],
                      preferred_element_type=jnp.float32)

        qb = lax.broadcasted_iota(jnp.int32, (Sq, Skv), 0) // BLK
        kb = lax.broadcasted_iota(jnp.int32, (Sq, Skv), 1) // BLK
        mask = (qb == kb) | (kb == 0) | ((qb + kb) % 3 == 0)

        for b in range(B):
            for h in range(H_LOC):
                qbh = q2d[b * Sq:(b + 1) * Sq, h * Dh:(h + 1) * Dh]
                kbh = k_ref[b, h]
                s = lax.dot_general(
                    qbh, kbh, (((1,), (1,)), ((), ())),
                    preferred_element_type=jnp.float32,
                ) * 0.125
                s = jnp.where(mask, s, -1e9)
                m = jnp.max(s, axis=1, keepdims=True)
                w = jnp.exp(s - m)
                w = w / jnp.sum(w, axis=1, keepdims=True)
                ctx_ref[b * Sq:(b + 1) * Sq, h * Dh:(h + 1) * Dh] = jnp.dot(
                    w, v_ref[b, h], preferred_element_type=jnp.float32)

        partial = jnp.dot(ctx_ref[:, :], wo_ref[:, :],
                          preferred_element_type=jnp.float32)
        comm_ref[0] = partial
        out_ref[:, :] = partial

        for hop in range(N_DEV - 1):
            rdma = pltpu.make_async_remote_copy(
                src_ref=comm_ref.at[hop],
                dst_ref=comm_ref.at[hop + 1],
                send_sem=send_sems.at[hop],
                recv_sem=recv_sems.at[hop],
                device_id=(right,),
                device_id_type=pl.DeviceIdType.MESH,
            )
            rdma.start()
            rdma.wait()
            out_ref[:, :] = out_ref[:, :] + comm_ref[hop + 1]

    out2d = pl.pallas_call(
        body,
        out_shape=jax.ShapeDtypeStruct((B * Sq, D_MODEL), jnp.float32),
        in_specs=[pl.BlockSpec(memory_space=pltpu.VMEM)] * 5,
        out_specs=pl.BlockSpec(memory_space=pltpu.VMEM),
        scratch_shapes=[
            pltpu.VMEM((B * Sq, H_LOC * Dh), jnp.float32),
            pltpu.VMEM((N_DEV, B * Sq, D_MODEL), jnp.float32),
            pltpu.SemaphoreType.DMA((N_DEV - 1,)),
            pltpu.SemaphoreType.DMA((N_DEV - 1,)),
        ],
        compiler_params=pltpu.CompilerParams(collective_id=0),
    )(x2d, Wq, Kh, Vh, Wo)
    return out2d.reshape(B, Sq, D_MODEL)


# device time: 24772 ns/iter; 1.9676x vs baseline; 1.9676x over previous
import jax
import jax.numpy as jnp
from jax import lax
from jax.experimental import pallas as pl
from jax.experimental.pallas import tpu as pltpu

N_DEV = 4
B, Sq, Skv, Dh = 2, 256, 256, 64
H_LOC = 4
D_MODEL = 512
BLK = 64


def kernel(x, Wq, K_ext, V_ext, Wo):
    my = lax.axis_index("i")
    Kh = lax.dynamic_slice_in_dim(K_ext, my * H_LOC, H_LOC, axis=2)
    Vh = lax.dynamic_slice_in_dim(V_ext, my * H_LOC, H_LOC, axis=2)
    Kh = jnp.transpose(Kh, (0, 2, 1, 3))
    Vh = jnp.transpose(Vh, (0, 2, 1, 3))
    x2d = x.reshape(B * Sq, D_MODEL)

    def body(x_ref, wq_ref, k_ref, v_ref, wo_ref, out_ref,
             ctx_ref, rs1_ref, rs2_ref, send_sems, recv_sems):
        pos = lax.axis_index("i")
        bit0 = jnp.bitwise_and(pos, 1)
        bit1 = pos // 2
        nb1 = jnp.bitwise_xor(pos, 1)
        nb2 = 3 - pos
        k1A = jnp.bitwise_xor(bit0, bit1)
        k2A = bit1
        k1B = bit1
        k2B = bit0

        barrier_sem = pltpu.get_barrier_semaphore()
        for nbr in (nb1, nb2):
            pl.semaphore_signal(
                barrier_sem, inc=1,
                device_id=(nbr,), device_id_type=pl.DeviceIdType.MESH,
            )
        pl.semaphore_wait(barrier_sem, 2)

        q2d = jnp.dot(x_ref[:, :], wq_ref[:, :],
                      preferred_element_type=jnp.float32)

        qb = lax.broadcasted_iota(jnp.int32, (Sq, Skv), 0) // BLK
        kb = lax.broadcasted_iota(jnp.int32, (Sq, Skv), 1) // BLK
        mask = (qb == kb) | (kb == 0) | ((qb + kb) % 3 == 0)

        def attn_batch(b):
            for h in range(H_LOC):
                qbh = q2d[b * Sq:(b + 1) * Sq, h * Dh:(h + 1) * Dh]
                s = lax.dot_general(
                    qbh, k_ref[b, h], (((1,), (1,)), ((), ())),
                    preferred_element_type=jnp.float32,
                ) * 0.125
                s = jnp.where(mask, s, -1e9)
                m = jnp.max(s, axis=1, keepdims=True)
                w = jnp.exp(s - m)
                w = w / jnp.sum(w, axis=1, keepdims=True)
                ctx_ref[b * Sq:(b + 1) * Sq, h * Dh:(h + 1) * Dh] = jnp.dot(
                    w, v_ref[b, h], preferred_element_type=jnp.float32)
            out_ref[b * Sq:(b + 1) * Sq, :] = jnp.dot(
                ctx_ref[b * Sq:(b + 1) * Sq, :], wo_ref[:, :],
                preferred_element_type=jnp.float32)

        def xchg(idx, src_slice, dst_ref_sliced, dev):
            return pltpu.make_async_remote_copy(
                src_ref=src_slice,
                dst_ref=dst_ref_sliced,
                send_sem=send_sems.at[idx],
                recv_sem=recv_sems.at[idx],
                device_id=(dev,),
                device_id_type=pl.DeviceIdType.MESH,
            )

        attn_batch(0)
        a1_keep = k1A * 128
        A1 = xchg(0, out_ref.at[pl.ds((1 - k1A) * 128, 128), :],
                  rs1_ref.at[0], nb1)
        A1.start()

        attn_batch(1)
        b1_keep = 256 + k1B * 128
        B1 = xchg(1, out_ref.at[pl.ds(256 + (1 - k1B) * 128, 128), :],
                  rs1_ref.at[1], nb2)
        B1.start()

        A1.wait()
        out_ref[pl.ds(a1_keep, 128), :] = (
            out_ref[pl.ds(a1_keep, 128), :] + rs1_ref[0])
        a2_keep = a1_keep + k2A * 64
        A2 = xchg(2, out_ref.at[pl.ds(a1_keep + (1 - k2A) * 64, 64), :],
                  rs2_ref.at[0], nb2)
        A2.start()

        B1.wait()
        out_ref[pl.ds(b1_keep, 128), :] = (
            out_ref[pl.ds(b1_keep, 128), :] + rs1_ref[1])
        b2_keep = b1_keep + k2B * 64
        B2 = xchg(3, out_ref.at[pl.ds(b1_keep + (1 - k2B) * 64, 64), :],
                  rs2_ref.at[1], nb1)
        B2.start()

        A2.wait()
        out_ref[pl.ds(a2_keep, 64), :] = (
            out_ref[pl.ds(a2_keep, 64), :] + rs2_ref[0])
        A3 = xchg(4, out_ref.at[pl.ds(a2_keep, 64), :],
                  out_ref.at[pl.ds(a2_keep, 64), :], nb2)
        A3.start()

        B2.wait()
        out_ref[pl.ds(b2_keep, 64), :] = (
            out_ref[pl.ds(b2_keep, 64), :] + rs2_ref[1])
        B3 = xchg(5, out_ref.at[pl.ds(b2_keep, 64), :],
                  out_ref.at[pl.ds(b2_keep, 64), :], nb1)
        B3.start()

        A3.wait()
        A4 = xchg(6, out_ref.at[pl.ds(a1_keep, 128), :],
                  out_ref.at[pl.ds(a1_keep, 128), :], nb1)
        A4.start()

        B3.wait()
        B4 = xchg(7, out_ref.at[pl.ds(b1_keep, 128), :],
                  out_ref.at[pl.ds(b1_keep, 128), :], nb2)
        B4.start()

        A4.wait()
        B4.wait()

    out2d = pl.pallas_call(
        body,
        out_shape=jax.ShapeDtypeStruct((B * Sq, D_MODEL), jnp.float32),
        in_specs=[pl.BlockSpec(memory_space=pltpu.VMEM)] * 5,
        out_specs=pl.BlockSpec(memory_space=pltpu.VMEM),
        scratch_shapes=[
            pltpu.VMEM((B * Sq, H_LOC * Dh), jnp.float32),
            pltpu.VMEM((2, 128, D_MODEL), jnp.float32),
            pltpu.VMEM((2, 64, D_MODEL), jnp.float32),
            pltpu.SemaphoreType.DMA((8,)),
            pltpu.SemaphoreType.DMA((8,)),
        ],
        compiler_params=pltpu.CompilerParams(collective_id=0),
    )(x2d, Wq, Kh, Vh, Wo)
    return out2d.reshape(B, Sq, D_MODEL)


# device time: 18555 ns/iter; 2.6268x vs baseline; 1.3351x over previous
import jax
import jax.numpy as jnp
from jax import lax
from jax.experimental import pallas as pl
from jax.experimental.pallas import tpu as pltpu

N_DEV = 4
B, Sq, Skv, Dh = 2, 256, 256, 64
H_LOC = 4
D_MODEL = 512
BLK = 64
HALF = B * Sq // 2


def kernel(x, Wq, K_ext, V_ext, Wo):
    my = lax.axis_index("i")
    Kh = lax.dynamic_slice_in_dim(K_ext, my * H_LOC, H_LOC, axis=2)
    Vh = lax.dynamic_slice_in_dim(V_ext, my * H_LOC, H_LOC, axis=2)
    Kh = jnp.transpose(Kh, (0, 2, 1, 3))
    Vh = jnp.transpose(Vh, (0, 2, 1, 3))
    x2d = x.reshape(B * Sq, D_MODEL)

    def body(x_ref, wq_ref, k_ref, v_ref, wo_ref, out_ref,
             ctx_ref, sbuf_ref, rbuf_ref, send_sems, recv_sems):
        pos = lax.axis_index("i")
        nb1 = jnp.bitwise_xor(pos, 1)
        nb2 = 3 - pos

        barrier_sem = pltpu.get_barrier_semaphore()
        for nbr in (nb1, nb2):
            pl.semaphore_signal(
                barrier_sem, inc=1,
                device_id=(nbr,), device_id_type=pl.DeviceIdType.MESH,
            )
        pl.semaphore_wait(barrier_sem, 2)

        q2d = jnp.dot(x_ref[:, :], wq_ref[:, :],
                      preferred_element_type=jnp.float32) * 0.125

        qb = lax.broadcasted_iota(jnp.int32, (Sq, Skv), 0) // BLK
        kb = lax.broadcasted_iota(jnp.int32, (Sq, Skv), 1) // BLK
        mask = (qb == kb) | (kb == 0) | ((qb + kb) % 3 == 0)

        def attn_batch(b):
            for h in range(H_LOC):
                qbh = q2d[b * Sq:(b + 1) * Sq, h * Dh:(h + 1) * Dh]
                s = lax.dot_general(
                    qbh, k_ref[b, h], (((1,), (1,)), ((), ())),
                    preferred_element_type=jnp.float32,
                )
                s = jnp.where(mask, s, -1e9)
                m = jnp.max(s, axis=1, keepdims=True)
                w = jnp.exp(s - m)
                w = w / jnp.sum(w, axis=1, keepdims=True)
                ctx_ref[b * Sq:(b + 1) * Sq, h * Dh:(h + 1) * Dh] = jnp.dot(
                    w, v_ref[b, h], preferred_element_type=jnp.float32)
            out_ref[b * Sq:(b + 1) * Sq, :] = jnp.dot(
                ctx_ref[b * Sq:(b + 1) * Sq, :], wo_ref[:, :],
                preferred_element_type=jnp.float32)

        def xchg(idx, half, dev):
            sbuf_ref[idx] = out_ref[pl.ds(half * HALF, HALF), :].astype(
                jnp.bfloat16)
            rdma = pltpu.make_async_remote_copy(
                src_ref=sbuf_ref.at[idx],
                dst_ref=rbuf_ref.at[idx],
                send_sem=send_sems.at[idx],
                recv_sem=recv_sems.at[idx],
                device_id=(dev,),
                device_id_type=pl.DeviceIdType.MESH,
            )
            rdma.start()
            return rdma

        def absorb(rdma, idx, half):
            rdma.wait()
            out_ref[pl.ds(half * HALF, HALF), :] = (
                out_ref[pl.ds(half * HALF, HALF), :]
                + rbuf_ref[idx].astype(jnp.float32))

        attn_batch(0)
        A1 = xchg(0, 0, nb1)
        attn_batch(1)
        B1 = xchg(1, 1, nb2)
        absorb(A1, 0, 0)
        A2 = xchg(2, 0, nb2)
        absorb(B1, 1, 1)
        B2 = xchg(3, 1, nb1)
        absorb(A2, 2, 0)
        absorb(B2, 3, 1)

    out2d = pl.pallas_call(
        body,
        out_shape=jax.ShapeDtypeStruct((B * Sq, D_MODEL), jnp.float32),
        in_specs=[pl.BlockSpec(memory_space=pltpu.VMEM)] * 5,
        out_specs=pl.BlockSpec(memory_space=pltpu.VMEM),
        scratch_shapes=[
            pltpu.VMEM((B * Sq, H_LOC * Dh), jnp.float32),
            pltpu.VMEM((4, HALF, D_MODEL), jnp.bfloat16),
            pltpu.VMEM((4, HALF, D_MODEL), jnp.bfloat16),
            pltpu.SemaphoreType.DMA((4,)),
            pltpu.SemaphoreType.DMA((4,)),
        ],
        compiler_params=pltpu.CompilerParams(collective_id=0),
    )(x2d, Wq, Kh, Vh, Wo)
    return out2d.reshape(B, Sq, D_MODEL)
